# Initial kernel scaffold; baseline (speedup 1.0000x reference)
import jax
import jax.numpy as jnp
from jax.experimental import pallas as pl


def kernel(ad_feature_inputs, feature_inputs, tables, W_meta, b_meta, W_pred, b_pred):
    raise NotImplementedError("write your pallas kernel here")


# speedup vs baseline: 1.3600x; 1.3600x over previous
import functools

import jax
import jax.numpy as jnp
from jax import lax
from jax.experimental import pallas as pl
from jax.experimental.pallas import tpu as pltpu
from jax.experimental.pallas import tpu_sc as plsc

_NUM_FIELDS = 26
_VOCAB = 100000
_EMB = 32
_B = 4096
_NF = 33
_NC = 2
_NS = 16
_NW = _NC * _NS
_RPW = _B // _NW
_CHUNK = 64
_NCH = _RPW // _CHUNK
_HALF = 16


def _sc_body(g_hbm, t_hbm, w_hbm, bias_hbm, out_hbm,
             idx_v, buf_v, acc_v, outc_v, wv_v, bias_v, sem):
    wid = lax.axis_index("s") * _NC + lax.axis_index("c")
    pltpu.sync_copy(w_hbm, wv_v)
    pltpu.sync_copy(bias_hbm, bias_v)

    for c in range(_NCH):
        pltpu.sync_copy(g_hbm.at[wid, c], idx_v)

        def fire(j, carry):
            pltpu.async_copy(t_hbm.at[idx_v.at[j]],
                             buf_v.at[pl.ds(j * _CHUNK, _CHUNK)], sem)
            return carry

        lax.fori_loop(0, _NF, fire, 0)
        pltpu.make_async_copy(t_hbm.at[pl.ds(0, _NF * _CHUNK)], buf_v, sem).wait()

        for j in range(_NF):
            w_lo = wv_v[j, pl.ds(0, _HALF)]
            w_hi = wv_v[j, pl.ds(_HALF, _HALF)]

            def row_body(r, carry, j=j, w_lo=w_lo, w_hi=w_hi):
                lo = buf_v[j * _CHUNK + r, pl.ds(0, _HALF)] * w_lo
                hi = buf_v[j * _CHUNK + r, pl.ds(_HALF, _HALF)] * w_hi
                if j == 0:
                    acc_v[pl.ds(r * _EMB, _HALF)] = lo
                    acc_v[pl.ds(r * _EMB + _HALF, _HALF)] = hi
                else:
                    plsc.addupdate(acc_v.at[pl.ds(r * _EMB, _HALF)], lo)
                    plsc.addupdate(acc_v.at[pl.ds(r * _EMB + _HALF, _HALF)], hi)
                return carry

            lax.fori_loop(0, _CHUNK, row_body, 0)

        bias = bias_v[:]
        for r16 in range(_CHUNK // _HALF):
            ridx = (lax.iota(jnp.int32, _HALF) + r16 * _HALF) * _EMB
            tot = plsc.load_gather(acc_v, [ridx])
            for k in range(1, _EMB):
                tot = tot + plsc.load_gather(acc_v, [ridx + k])
            tot = tot + bias
            p = 1.0 / (1.0 + jnp.exp(-tot))
            outc_v[pl.ds(c * _CHUNK + r16 * _HALF, _HALF)] = p

    pltpu.sync_copy(outc_v, out_hbm.at[pl.ds(wid * _RPW, _RPW)])


@functools.partial(jax.jit, static_argnums=())
def kernel(ad_feature_inputs, feature_inputs, tables, W_meta, b_meta,
           W_pred, b_pred):
    t_flat = tables.reshape(_NUM_FIELDS * _VOCAB, _EMB)

    w0 = W_pred[0, :_EMB]
    v = W_meta.T @ w0
    w_eff = jnp.concatenate(
        [W_pred[0, _EMB:].reshape(_NF - 8, _EMB),
         jnp.broadcast_to((v / _EMB)[:, None], (8, _EMB))], axis=0)
    bias_c = b_pred[0] + jnp.dot(b_meta, w0)
    bias_vec = jnp.full((_HALF,), bias_c, jnp.float32)

    offs_f = (jnp.arange(_NF - 8, dtype=jnp.int32) + 1) * _VOCAB
    offs_a = (jnp.arange(8, dtype=jnp.int32) + 1) * _VOCAB
    g = jnp.concatenate([feature_inputs + offs_f[None, :],
                         ad_feature_inputs + offs_a[None, :]], axis=1)
    g = g.reshape(_NW, _NCH, _CHUNK, _NF).transpose(0, 1, 3, 2)

    mesh = plsc.VectorSubcoreMesh(core_axis_name="c", subcore_axis_name="s")
    out = pl.kernel(
        _sc_body,
        out_type=jax.ShapeDtypeStruct((_B,), jnp.float32),
        mesh=mesh,
        compiler_params=pltpu.CompilerParams(needs_layout_passes=False,
                                             use_tc_tiling_on_sc=False),
        scratch_types=[
            pltpu.VMEM((_NF, _CHUNK), jnp.int32),
            pltpu.VMEM((_NF * _CHUNK, _EMB), jnp.float32),
            pltpu.VMEM((_CHUNK * _EMB,), jnp.float32),
            pltpu.VMEM((_RPW,), jnp.float32),
            pltpu.VMEM((_NF, _EMB), jnp.float32),
            pltpu.VMEM((_HALF,), jnp.float32),
            pltpu.SemaphoreType.DMA,
        ],
    )(g, t_flat, w_eff, bias_vec)
    return out[:, None]
